# baseline (speedup 1.0000x reference)
import functools

import jax
import jax.numpy as jnp
from jax import lax
from jax.experimental import pallas as pl
from jax.experimental.pallas import tpu as pltpu
from jax.experimental.pallas import tpu_sc as plsc

N_USERS = 10000
M_ITEMS = 40000
N = 50000
D = 64
P = 4
E_S = 160000
E_SUB = 200000
B = 4096

NC = 2
NS = 16
K = 128

HALF_L = 25000
HALF_P = 25088
NP_ROWS = 2 * HALF_P
SHALF_L = 5000
SHALF_P = 5120
SNP_ROWS = 2 * SHALF_P

E_SUB_PAD = 202752
E_S_PAD = 165888

_MESH = plsc.VectorSubcoreMesh(core_axis_name="c", subcore_axis_name="s")


def _pad_idx(col, thresh, pad):
    if pad == 0:
        return col
    return jnp.where(col >= thresh, col + pad, col)


def _make_spmm(n_src, src_thresh, src_pad, half_l, half_p, e_pad, reps=1):
    e_tile = e_pad // NS
    n_chunks = e_tile // K
    acc_rows = reps * half_p + NS * K
    tile_rows = half_p // NS
    ztotal = reps * half_p // NS
    zrows = max(d for d in range(1, 57)
                if ztotal % d == 0 and tile_rows % d == 0)

    @functools.partial(
        pl.kernel,
        mesh=_MESH,
        compiler_params=pltpu.CompilerParams(use_tc_tiling_on_sc=False,
                                             needs_layout_passes=False),
        out_type=jax.ShapeDtypeStruct((2 * half_p, D), jnp.float32),
        scratch_types=[
            pltpu.VMEM((3, K), jnp.int32),
            pltpu.VMEM((3, K), jnp.int32),
            pltpu.VMEM((K,), jnp.int32),
            pltpu.VMEM((K,), jnp.int32),
            pltpu.VMEM((K,), jnp.int32),
            pltpu.VMEM((K,), jnp.int32),
            pltpu.VMEM((K, D), jnp.float32),
            pltpu.VMEM((K, D), jnp.float32),
            pltpu.VMEM((zrows, D), jnp.float32),
            pltpu.VMEM((zrows if reps > 1 else 8, D), jnp.float32),
            pltpu.VMEM_SHARED((acc_rows, D), jnp.float32),
            pltpu.SemaphoreType.DMA,
            pltpu.SemaphoreType.DMA,
            pltpu.SemaphoreType.DMA,
            pltpu.SemaphoreType.DMA,
            pltpu.SemaphoreType.DMA,
            pltpu.SemaphoreType.DMA,
        ],
    )
    def spmm(pk_hbm, x_hbm, out_hbm,
             pk0, pk1, gi0, gi1, si0, si1, xr0, xr1, zero_v, fl_v, acc,
             p0, p1, g0, g1, s0, s1):
        c = lax.axis_index("c")
        s = lax.axis_index("s")
        pk_b = (pk0, pk1)
        gi_b = (gi0, gi1)
        si_b = (si0, si1)
        xr_b = (xr0, xr1)
        psem = (p0, p1)
        gsem = (g0, g1)
        ssem = (s0, s1)

        def _zb(i, carry):
            z = jnp.zeros((16,), jnp.float32)
            for q in range(4):
                zero_v[i, pl.ds(q * 16, 16)] = z
            return carry
        lax.fori_loop(0, zrows, _zb, 0)
        for k in range(ztotal // zrows):
            pltpu.sync_copy(zero_v,
                            acc.at[pl.ds(s * ztotal + k * zrows, zrows)])
        plsc.subcore_barrier()

        def _pk_copy(j, b):
            return pltpu.make_async_copy(pk_hbm.at[s * n_chunks + j],
                                         pk_b[b], psem[b])

        def _idx(b):
            pk = pk_b[b]
            lane = lax.iota(jnp.int32, 16)
            for g in range(K // 16):
                sl = pl.ds(g * 16, 16)
                cg = pk[1, sl]
                gi_b[b][sl] = _pad_idx(cg, src_thresh, src_pad)
                rg = pk[0, sl]
                vg = pk[2, sl]
                loc = rg - c * half_l
                ok = (loc >= 0) & (loc < half_l) & (vg != 0)
                if reps > 1:
                    loc = loc + (lane & (reps - 1)) * half_p
                trash = reps * half_p + s * K + g * 16 + lane
                si_b[b][sl] = jnp.where(ok, loc, trash)

        def _gather(b):
            return pltpu.make_async_copy(x_hbm.at[gi_b[b]], xr_b[b], gsem[b])

        def _scatter(b):
            return pltpu.make_async_copy(xr_b[b], acc.at[si_b[b]], ssem[b])

        def _scale(b):
            pk = pk_b[b]
            xr = xr_b[b]
            for g in range(K // 16):
                vv = lax.bitcast_convert_type(pk[2, pl.ds(g * 16, 16)],
                                              jnp.float32)
                for l in range(16):
                    v = vv[l]
                    e = g * 16 + l
                    for q in range(4):
                        sl = pl.ds(q * 16, 16)
                        xr[e, sl] = xr[e, sl] * v

        _pk_copy(0, 0).start()
        _pk_copy(0, 0).wait()
        _idx(0)
        _gather(0).start()
        _pk_copy(1, 1).start()

        def _body(i, carry):
            for b in range(2):
                j = 2 * i + b
                nb = 1 - b

                @pl.when(j < n_chunks)
                def _():
                    _gather(b).wait()
                    _scale(b)

                    @pl.when((j >= 1) & (j + 1 < n_chunks))
                    def _():
                        _scatter(nb).wait()

                    @pl.when(j + 1 < n_chunks)
                    def _():
                        _pk_copy(j + 1, nb).wait()
                        _idx(nb)
                        _gather(nb).start()

                    @pl.when(j + 2 < n_chunks)
                    def _():
                        _pk_copy(j + 2, b).start()
                    pltpu.async_copy(xr_b[b], acc.at[si_b[b]], ssem[b],
                                     add=True)
            return carry
        lax.fori_loop(0, (n_chunks + 1) // 2, _body, 0)
        _scatter(0).wait()
        _scatter(1).wait()
        plsc.subcore_barrier()

        if reps == 1:
            pltpu.sync_copy(acc.at[pl.ds(s * tile_rows, tile_rows)],
                            out_hbm.at[pl.ds(c * half_p + s * tile_rows,
                                             tile_rows)])
        else:
            for k in range(tile_rows // zrows):
                base = s * tile_rows + k * zrows
                pltpu.sync_copy(acc.at[pl.ds(base, zrows)], zero_v)
                for r in range(1, reps):
                    pltpu.sync_copy(acc.at[pl.ds(r * half_p + base, zrows)],
                                    fl_v)

                    def _addf(i, carry):
                        for q in range(4):
                            sl = pl.ds(q * 16, 16)
                            zero_v[i, sl] = zero_v[i, sl] + fl_v[i, sl]
                        return carry
                    lax.fori_loop(0, zrows, _addf, 0)
                pltpu.sync_copy(zero_v,
                                out_hbm.at[pl.ds(c * half_p + base, zrows)])

    return spmm


_spmm_sub = _make_spmm(NP_ROWS, HALF_L, 88, HALF_L, HALF_P, E_SUB_PAD)
_spmm_soc1 = _make_spmm(N_USERS, N_USERS, 0, SHALF_L, SHALF_P, E_S_PAD,
                        reps=4)
_spmm_soc2 = _make_spmm(SNP_ROWS, SHALF_L, 120, SHALF_L, SHALF_P, E_S_PAD,
                        reps=4)


def _pack_edges(rows, cols, vals, e_pad, e):
    rows = jnp.pad(rows.astype(jnp.int32), (0, e_pad - e)).reshape(-1, K)
    cols = jnp.pad(cols.astype(jnp.int32), (0, e_pad - e)).reshape(-1, K)
    vals = lax.bitcast_convert_type(jnp.pad(vals, (0, e_pad - e)),
                                    jnp.int32).reshape(-1, K)
    return jnp.stack([rows, cols, vals], axis=1)

FR = 2000


def _fusion_body(x0, x1, x2, x3, y, w1, b1, w2, b2, w3, b3, t3_ref, ssq_ref):
    x = x0[...] + x1[...] + x2[...] + x3[...]
    yv = y[...]
    c = jnp.concatenate([x, yv, x * yv], axis=1)
    dn = (((1,), (1,)), ((), ()))
    t1 = jnp.tanh(lax.dot_general(c, w1[...], dn,
                                  preferred_element_type=jnp.float32) + b1[...])
    t2 = jnp.tanh(lax.dot_general(t1, w2[...], dn,
                                  preferred_element_type=jnp.float32) + b2[...])
    t3 = lax.dot_general(t2, w3[...], dn,
                         preferred_element_type=jnp.float32) + b3[...]
    t3_ref[...] = t3

    @pl.when(pl.program_id(0) == 0)
    def _():
        ssq_ref[...] = jnp.zeros((1, 1), jnp.float32)
    ssq_ref[...] = ssq_ref[...] + jnp.sum(t3 * t3).reshape(1, 1)


def _fusion(x_parts, y, f1_W, f1_b, f2_W, f2_b, f3_W, f3_b):
    row_spec = pl.BlockSpec((FR, D), lambda i: (i, 0))
    full = lambda shape: pl.BlockSpec(shape, lambda i: (0,) * len(shape))
    t3, ssq = pl.pallas_call(
        _fusion_body,
        grid=(N_USERS // FR,),
        in_specs=[row_spec, row_spec, row_spec, row_spec, row_spec,
                  full((3 * D, 3 * D)), full((1, 3 * D)),
                  full((D, 3 * D)), full((1, D)),
                  full((D, D)), full((1, D))],
        out_specs=[row_spec, pl.BlockSpec((1, 1), lambda i: (0, 0))],
        out_shape=[jax.ShapeDtypeStruct((N_USERS, D), jnp.float32),
                   jax.ShapeDtypeStruct((1, 1), jnp.float32)],
    )(x_parts[0], x_parts[1], x_parts[2], x_parts[3], y,
      f1_W, f1_b.reshape(1, 3 * D), f2_W, f2_b.reshape(1, D),
      f3_W, f3_b.reshape(1, D))
    return t3, ssq


BT = B // (NC * NS)


@functools.partial(
    pl.kernel,
    mesh=_MESH,
    compiler_params=pltpu.CompilerParams(use_tc_tiling_on_sc=False, needs_layout_passes=False),
    out_type=jax.ShapeDtypeStruct((B,), jnp.float32),
    scratch_types=[
        pltpu.VMEM((BT,), jnp.int32),
        pltpu.VMEM((BT,), jnp.int32),
        pltpu.VMEM((BT,), jnp.int32),
        pltpu.VMEM((16,), jnp.float32),
        pltpu.VMEM((BT, D), jnp.float32),
        pltpu.VMEM((BT, D), jnp.float32),
        pltpu.VMEM((BT, D), jnp.float32),
        pltpu.VMEM((BT, D), jnp.float32),
        pltpu.VMEM((BT,), jnp.float32),
        pltpu.SemaphoreType.DMA,
    ],
)
def _final_sc(users_hbm, items_hbm, u_emb, i_emb, t31, t32,
              x10, x11, x12, x13, x20, x21, x22, x23, scal_hbm, out_hbm,
              uid_v, iid_v, nid_v, scal_v, urow_v, irow_v, g1_v, g2_v,
              gam_v, sem):
    c = lax.axis_index("c")
    s = lax.axis_index("s")
    wid = s * NC + c
    base = wid * BT
    pltpu.sync_copy(scal_hbm, scal_v)
    pltpu.sync_copy(users_hbm.at[pl.ds(base, BT)], uid_v)
    pltpu.sync_copy(items_hbm.at[pl.ds(base, BT)], iid_v)
    for g in range(BT // 16):
        sl = pl.ds(g * 16, 16)
        nid = iid_v[sl] + N_USERS
        nid_v[sl] = jnp.where(nid >= HALF_L, nid + 88, nid)
    scal16 = scal_v[pl.ds(0, 16)]
    c0 = scal16[0]
    c1 = scal16[1]
    c2 = scal16[2]
    c3 = scal16[3]

    pltpu.async_copy(u_emb.at[uid_v], urow_v, sem).wait()
    pltpu.async_copy(t31.at[uid_v], g1_v, sem).wait()
    pltpu.async_copy(t32.at[uid_v], g2_v, sem).wait()

    def _ucomb(i, carry):
        e = i // 4
        sl = pl.ds((i % 4) * 16, 16)
        urow_v[e, sl] = (urow_v[e, sl] * c0 + g1_v[e, sl] * c1
                         + g2_v[e, sl] * c2)
        return carry
    lax.fori_loop(0, BT * 4, _ucomb, 0)

    pltpu.async_copy(i_emb.at[iid_v], irow_v, sem).wait()

    def _iscale(i, carry):
        e = i // 4
        sl = pl.ds((i % 4) * 16, 16)
        irow_v[e, sl] = irow_v[e, sl] * c0
        return carry
    lax.fori_loop(0, BT * 4, _iscale, 0)
    for xp in (x10, x11, x12, x13, x20, x21, x22, x23):
        pltpu.async_copy(xp.at[nid_v], g1_v, sem).wait()

        def _iacc(i, carry):
            e = i // 4
            sl = pl.ds((i % 4) * 16, 16)
            irow_v[e, sl] = irow_v[e, sl] + g1_v[e, sl] * c3
            return carry
        lax.fori_loop(0, BT * 4, _iacc, 0)

    iota = lax.iota(jnp.int32, 16)

    def _dot(pg, carry):
        r = pg * 16 + iota
        acc = jnp.zeros((16,), jnp.float32)
        for d in range(D):
            cd = jnp.full((16,), d, jnp.int32)
            uvec = plsc.load_gather(urow_v, [r, cd])
            ivec = plsc.load_gather(irow_v, [r, cd])
            acc = acc + uvec * ivec
        gam_v[pl.ds(pg * 16, 16)] = acc
        return carry
    lax.fori_loop(0, BT // 16, _dot, 0)
    pltpu.sync_copy(gam_v, out_hbm.at[pl.ds(base, BT)])


def _pad50(x):
    z = jnp.zeros((HALF_P - HALF_L, D), jnp.float32)
    return jnp.concatenate([x[:HALF_L], z, x[HALF_L:], z], axis=0)


def kernel(users_, items_, U_emb, I_emb, inter_row, inter_col, inter_val,
           social_row, social_col, social_val, sub_rows, sub_cols, sub_vals,
           fc_W, fc_b, fc2_W, fc2_b, fcp_W, fcp_b,
           f1_W, f1_b, f2_W, f2_b, f3_W, f3_b):
    e0p = _pad50(jnp.concatenate([U_emb, I_emb], axis=0))

    pk_sub = [_pack_edges(sub_rows[p], sub_cols[p], sub_vals[p],
                          E_SUB_PAD, E_SUB) for p in range(P)]
    pk_soc = _pack_edges(social_row, social_col, social_val, E_S_PAD, E_S)

    x1 = [_spmm_sub(pk_sub[p], e0p) for p in range(P)]
    us1 = _spmm_soc1(pk_soc, U_emb)
    us1_u = jnp.concatenate([us1[:SHALF_L], us1[SHALF_P:SHALF_P + SHALF_L]])
    t31, ssq1 = _fusion(x1, us1_u, f1_W, f1_b, f2_W, f2_b, f3_W, f3_b)

    x2 = [_spmm_sub(pk_sub[p], x1[p]) for p in range(P)]
    us2 = _spmm_soc2(pk_soc, us1)
    us2_u = jnp.concatenate([us2[:SHALF_L], us2[SHALF_P:SHALF_P + SHALF_L]])
    t32, ssq2 = _fusion(x2, us2_u, f1_W, f1_b, f2_W, f2_b, f3_W, f3_b)

    inv1 = lax.rsqrt(ssq1[0, 0])
    inv2 = lax.rsqrt(ssq2[0, 0])
    scal = jnp.zeros((16,), jnp.float32)
    scal = scal.at[0].set(4.0 / 3.0)
    scal = scal.at[1].set(inv1 / 3.0)
    scal = scal.at[2].set(inv2 / 3.0)
    scal = scal.at[3].set(1.0 / 3.0)

    gamma = _final_sc(users_.astype(jnp.int32), items_.astype(jnp.int32),
                      U_emb, I_emb, t31, t32,
                      x1[0], x1[1], x1[2], x1[3],
                      x2[0], x2[1], x2[2], x2[3], scal)
    return gamma


# speedup vs baseline: 1.0405x; 1.0405x over previous
import functools

import jax
import jax.numpy as jnp
from jax import lax
from jax.experimental import pallas as pl
from jax.experimental.pallas import tpu as pltpu
from jax.experimental.pallas import tpu_sc as plsc

N_USERS = 10000
M_ITEMS = 40000
N = 50000
D = 64
P = 4
E_S = 160000
E_SUB = 200000
B = 4096

NC = 2
NS = 16
K = 128

HALF_L = 25000
HALF_P = 25088
NP_ROWS = 2 * HALF_P
SHALF_L = 5000
SHALF_P = 5120
SNP_ROWS = 2 * SHALF_P

E_SUB_PAD = 200704
E_S_PAD = 161792

_MESH = plsc.VectorSubcoreMesh(core_axis_name="c", subcore_axis_name="s")


def _pad_idx(col, thresh, pad):
    if pad == 0:
        return col
    return jnp.where(col >= thresh, col + pad, col)


def _make_spmm(n_src, src_thresh, src_pad, half_l, half_p, e_pad,
               zero_chunks, zero_rows):
    e_tile = e_pad // NS
    n_chunks = e_tile // K
    acc_rows = half_p + 8

    @functools.partial(
        pl.kernel,
        mesh=_MESH,
        compiler_params=pltpu.CompilerParams(use_tc_tiling_on_sc=False,
                                             needs_layout_passes=False),
        out_type=jax.ShapeDtypeStruct((2 * half_p, D), jnp.float32),
        scratch_types=[
            pltpu.VMEM((3, K), jnp.int32),
            pltpu.VMEM((K,), jnp.int32),
            pltpu.VMEM((K,), jnp.int32),
            pltpu.VMEM((K, D), jnp.float32),
            pltpu.VMEM((zero_rows, D), jnp.float32),
            pltpu.VMEM_SHARED((acc_rows, D), jnp.float32),
            pltpu.SemaphoreType.DMA,
        ],
    )
    def spmm(pk_hbm, x_hbm, out_hbm,
             pk_v, gidx_v, sidx_v, xr_v, zero_v, acc, sem):
        c = lax.axis_index("c")
        s = lax.axis_index("s")

        def _zb(i, carry):
            z = jnp.zeros((16,), jnp.float32)
            for q in range(4):
                zero_v[i, pl.ds(q * 16, 16)] = z
            return carry
        lax.fori_loop(0, zero_rows, _zb, 0)
        tile_rows = zero_chunks * zero_rows
        for k in range(zero_chunks):
            pltpu.sync_copy(zero_v, acc.at[pl.ds(s * tile_rows + k * zero_rows,
                                                 zero_rows)])
        plsc.subcore_barrier()

        def _chunk(j, carry):
            pltpu.sync_copy(pk_hbm.at[s * n_chunks + j], pk_v)
            for g in range(K // 16):
                sl = pl.ds(g * 16, 16)
                cg = pk_v[1, sl]
                gidx_v[sl] = _pad_idx(cg, src_thresh, src_pad)
                rg = pk_v[0, sl]
                loc = rg - c * half_l
                ok = (loc >= 0) & (loc < half_l)
                sidx_v[sl] = jnp.where(ok, loc, half_p)
            pltpu.async_copy(x_hbm.at[gidx_v], xr_v, sem).wait()
            for g in range(K // 16):
                vv = lax.bitcast_convert_type(pk_v[2, pl.ds(g * 16, 16)],
                                              jnp.float32)
                for l in range(16):
                    v = vv[l]
                    e = g * 16 + l
                    for q in range(4):
                        sl = pl.ds(q * 16, 16)
                        xr_v[e, sl] = xr_v[e, sl] * v
            pltpu.sync_copy(xr_v, acc.at[sidx_v], add=True)
            return carry
        lax.fori_loop(0, n_chunks, _chunk, 0)
        plsc.subcore_barrier()

        for k in range(zero_chunks):
            r0 = s * tile_rows + k * zero_rows
            pltpu.sync_copy(acc.at[pl.ds(r0, zero_rows)],
                            out_hbm.at[pl.ds(c * half_p + r0, zero_rows)])

    return spmm


_spmm_sub = _make_spmm(NP_ROWS, HALF_L, 88, HALF_L, HALF_P, E_SUB_PAD, 28, 56)
_spmm_soc1 = _make_spmm(N_USERS, N_USERS, 0, SHALF_L, SHALF_P, E_S_PAD, 5, 64)
_spmm_soc2 = _make_spmm(SNP_ROWS, SHALF_L, 120, SHALF_L, SHALF_P, E_S_PAD, 5, 64)


def _pack_edges(rows, cols, vals, e_pad, e):
    rows = jnp.pad(rows.astype(jnp.int32), (0, e_pad - e)).reshape(-1, K)
    cols = jnp.pad(cols.astype(jnp.int32), (0, e_pad - e)).reshape(-1, K)
    vals = lax.bitcast_convert_type(jnp.pad(vals, (0, e_pad - e)),
                                    jnp.int32).reshape(-1, K)
    return jnp.stack([rows, cols, vals], axis=1)

FR = 2000


def _fusion_body(x0, x1, x2, x3, y, w1, b1, w2, b2, w3, b3, t3_ref, ssq_ref):
    x = x0[...] + x1[...] + x2[...] + x3[...]
    yv = y[...]
    c = jnp.concatenate([x, yv, x * yv], axis=1)
    dn = (((1,), (1,)), ((), ()))
    t1 = jnp.tanh(lax.dot_general(c, w1[...], dn,
                                  preferred_element_type=jnp.float32) + b1[...])
    t2 = jnp.tanh(lax.dot_general(t1, w2[...], dn,
                                  preferred_element_type=jnp.float32) + b2[...])
    t3 = lax.dot_general(t2, w3[...], dn,
                         preferred_element_type=jnp.float32) + b3[...]
    t3_ref[...] = t3

    @pl.when(pl.program_id(0) == 0)
    def _():
        ssq_ref[...] = jnp.zeros((1, 1), jnp.float32)
    ssq_ref[...] = ssq_ref[...] + jnp.sum(t3 * t3).reshape(1, 1)


def _fusion(x_parts, y, f1_W, f1_b, f2_W, f2_b, f3_W, f3_b):
    row_spec = pl.BlockSpec((FR, D), lambda i: (i, 0))
    full = lambda shape: pl.BlockSpec(shape, lambda i: (0,) * len(shape))
    t3, ssq = pl.pallas_call(
        _fusion_body,
        grid=(N_USERS // FR,),
        in_specs=[row_spec, row_spec, row_spec, row_spec, row_spec,
                  full((3 * D, 3 * D)), full((1, 3 * D)),
                  full((D, 3 * D)), full((1, D)),
                  full((D, D)), full((1, D))],
        out_specs=[row_spec, pl.BlockSpec((1, 1), lambda i: (0, 0))],
        out_shape=[jax.ShapeDtypeStruct((N_USERS, D), jnp.float32),
                   jax.ShapeDtypeStruct((1, 1), jnp.float32)],
    )(x_parts[0], x_parts[1], x_parts[2], x_parts[3], y,
      f1_W, f1_b.reshape(1, 3 * D), f2_W, f2_b.reshape(1, D),
      f3_W, f3_b.reshape(1, D))
    return t3, ssq


BT = B // (NC * NS)


@functools.partial(
    pl.kernel,
    mesh=_MESH,
    compiler_params=pltpu.CompilerParams(use_tc_tiling_on_sc=False, needs_layout_passes=False),
    out_type=jax.ShapeDtypeStruct((B,), jnp.float32),
    scratch_types=[
        pltpu.VMEM((BT,), jnp.int32),
        pltpu.VMEM((BT,), jnp.int32),
        pltpu.VMEM((BT,), jnp.int32),
        pltpu.VMEM((16,), jnp.float32),
        pltpu.VMEM((BT, D), jnp.float32),
        pltpu.VMEM((BT, D), jnp.float32),
        pltpu.VMEM((BT, D), jnp.float32),
        pltpu.VMEM((BT, D), jnp.float32),
        pltpu.VMEM((BT,), jnp.float32),
        pltpu.SemaphoreType.DMA,
    ],
)
def _final_sc(users_hbm, items_hbm, u_emb, i_emb, t31, t32,
              x10, x11, x12, x13, x20, x21, x22, x23, scal_hbm, out_hbm,
              uid_v, iid_v, nid_v, scal_v, urow_v, irow_v, g1_v, g2_v,
              gam_v, sem):
    c = lax.axis_index("c")
    s = lax.axis_index("s")
    wid = s * NC + c
    base = wid * BT
    pltpu.sync_copy(scal_hbm, scal_v)
    pltpu.sync_copy(users_hbm.at[pl.ds(base, BT)], uid_v)
    pltpu.sync_copy(items_hbm.at[pl.ds(base, BT)], iid_v)
    for g in range(BT // 16):
        sl = pl.ds(g * 16, 16)
        nid = iid_v[sl] + N_USERS
        nid_v[sl] = jnp.where(nid >= HALF_L, nid + 88, nid)
    scal16 = scal_v[pl.ds(0, 16)]
    c0 = scal16[0]
    c1 = scal16[1]
    c2 = scal16[2]
    c3 = scal16[3]

    pltpu.async_copy(u_emb.at[uid_v], urow_v, sem).wait()
    pltpu.async_copy(t31.at[uid_v], g1_v, sem).wait()
    pltpu.async_copy(t32.at[uid_v], g2_v, sem).wait()

    def _ucomb(i, carry):
        e = i // 4
        sl = pl.ds((i % 4) * 16, 16)
        urow_v[e, sl] = (urow_v[e, sl] * c0 + g1_v[e, sl] * c1
                         + g2_v[e, sl] * c2)
        return carry
    lax.fori_loop(0, BT * 4, _ucomb, 0)

    pltpu.async_copy(i_emb.at[iid_v], irow_v, sem).wait()

    def _iscale(i, carry):
        e = i // 4
        sl = pl.ds((i % 4) * 16, 16)
        irow_v[e, sl] = irow_v[e, sl] * c0
        return carry
    lax.fori_loop(0, BT * 4, _iscale, 0)
    for xp in (x10, x11, x12, x13, x20, x21, x22, x23):
        pltpu.async_copy(xp.at[nid_v], g1_v, sem).wait()

        def _iacc(i, carry):
            e = i // 4
            sl = pl.ds((i % 4) * 16, 16)
            irow_v[e, sl] = irow_v[e, sl] + g1_v[e, sl] * c3
            return carry
        lax.fori_loop(0, BT * 4, _iacc, 0)

    iota = lax.iota(jnp.int32, 16)

    def _dot(pg, carry):
        r = pg * 16 + iota
        acc = jnp.zeros((16,), jnp.float32)
        for d in range(D):
            cd = jnp.full((16,), d, jnp.int32)
            uvec = plsc.load_gather(urow_v, [r, cd])
            ivec = plsc.load_gather(irow_v, [r, cd])
            acc = acc + uvec * ivec
        gam_v[pl.ds(pg * 16, 16)] = acc
        return carry
    lax.fori_loop(0, BT // 16, _dot, 0)
    pltpu.sync_copy(gam_v, out_hbm.at[pl.ds(base, BT)])


def _pad50(x):
    z = jnp.zeros((HALF_P - HALF_L, D), jnp.float32)
    return jnp.concatenate([x[:HALF_L], z, x[HALF_L:], z], axis=0)


def kernel(users_, items_, U_emb, I_emb, inter_row, inter_col, inter_val,
           social_row, social_col, social_val, sub_rows, sub_cols, sub_vals,
           fc_W, fc_b, fc2_W, fc2_b, fcp_W, fcp_b,
           f1_W, f1_b, f2_W, f2_b, f3_W, f3_b):
    e0p = _pad50(jnp.concatenate([U_emb, I_emb], axis=0))

    pk_sub = [_pack_edges(sub_rows[p], sub_cols[p], sub_vals[p],
                          E_SUB_PAD, E_SUB) for p in range(P)]
    pk_soc = _pack_edges(social_row, social_col, social_val, E_S_PAD, E_S)

    x1 = [_spmm_sub(pk_sub[p], e0p) for p in range(P)]
    us1 = _spmm_soc1(pk_soc, U_emb)
    us1_u = jnp.concatenate([us1[:SHALF_L], us1[SHALF_P:SHALF_P + SHALF_L]])
    t31, ssq1 = _fusion(x1, us1_u, f1_W, f1_b, f2_W, f2_b, f3_W, f3_b)

    x2 = [_spmm_sub(pk_sub[p], x1[p]) for p in range(P)]
    us2 = _spmm_soc2(pk_soc, us1)
    us2_u = jnp.concatenate([us2[:SHALF_L], us2[SHALF_P:SHALF_P + SHALF_L]])
    t32, ssq2 = _fusion(x2, us2_u, f1_W, f1_b, f2_W, f2_b, f3_W, f3_b)

    inv1 = lax.rsqrt(ssq1[0, 0])
    inv2 = lax.rsqrt(ssq2[0, 0])
    scal = jnp.zeros((16,), jnp.float32)
    scal = scal.at[0].set(4.0 / 3.0)
    scal = scal.at[1].set(inv1 / 3.0)
    scal = scal.at[2].set(inv2 / 3.0)
    scal = scal.at[3].set(1.0 / 3.0)

    gamma = _final_sc(users_.astype(jnp.int32), items_.astype(jnp.int32),
                      U_emb, I_emb, t31, t32,
                      x1[0], x1[1], x1[2], x1[3],
                      x2[0], x2[1], x2[2], x2[3], scal)
    return gamma


# speedup vs baseline: 1.2251x; 1.1774x over previous
import functools

import jax
import jax.numpy as jnp
from jax import lax
from jax.experimental import pallas as pl
from jax.experimental.pallas import tpu as pltpu
from jax.experimental.pallas import tpu_sc as plsc

N_USERS = 10000
M_ITEMS = 40000
N = 50000
D = 64
P = 4
E_S = 160000
E_SUB = 200000
B = 4096

NC = 2
NS = 16
K = 128

HALF_L = 25000
HALF_P = 25088
NP_ROWS = 2 * HALF_P
SHALF_L = 5000
SHALF_P = 5120
SNP_ROWS = 2 * SHALF_P

E_SUB_PAD = 200704
E_S_PAD = 161792

_MESH = plsc.VectorSubcoreMesh(core_axis_name="c", subcore_axis_name="s")


def _pad_idx(col, thresh, pad):
    if pad == 0:
        return col
    return jnp.where(col >= thresh, col + pad, col)


def _make_spmm(n_src, src_thresh, src_pad, half_l, half_p, e_pad,
               zero_chunks, zero_rows):
    e_tile = e_pad // NS
    n_chunks = e_tile // K
    acc_rows = half_p + 8

    @functools.partial(
        pl.kernel,
        mesh=_MESH,
        compiler_params=pltpu.CompilerParams(use_tc_tiling_on_sc=False,
                                             needs_layout_passes=False),
        out_type=jax.ShapeDtypeStruct((2 * half_p, D), jnp.float32),
        scratch_types=[
            pltpu.VMEM((3, K), jnp.int32),
            pltpu.VMEM((3, K), jnp.int32),
            pltpu.VMEM((K,), jnp.int32),
            pltpu.VMEM((K,), jnp.int32),
            pltpu.VMEM((K, D), jnp.float32),
            pltpu.VMEM((zero_rows, D), jnp.float32),
            pltpu.VMEM_SHARED((acc_rows, D), jnp.float32),
            pltpu.SemaphoreType.DMA,
            pltpu.SemaphoreType.DMA,
            pltpu.SemaphoreType.DMA,
        ],
    )
    def spmm(pk_hbm, x_hbm, out_hbm,
             pk0, pk1, gidx_v, sidx_v, xr_v, zero_v, acc, sem, q0, q1):
        c = lax.axis_index("c")
        s = lax.axis_index("s")

        def _zb(i, carry):
            z = jnp.zeros((16,), jnp.float32)
            for q in range(4):
                zero_v[i, pl.ds(q * 16, 16)] = z
            return carry
        lax.fori_loop(0, zero_rows, _zb, 0)
        tile_rows = zero_chunks * zero_rows
        for k in range(zero_chunks):
            pltpu.sync_copy(zero_v, acc.at[pl.ds(s * tile_rows + k * zero_rows,
                                                 zero_rows)])
        plsc.subcore_barrier()

        pk_b = (pk0, pk1)
        qsem = (q0, q1)

        def _pk_copy(j, b):
            return pltpu.make_async_copy(pk_hbm.at[s * n_chunks + j],
                                         pk_b[b], qsem[b])

        def _process(j, b):
            pk_v = pk_b[b]
            _pk_copy(j, b).wait()
            for g in range(K // 16):
                sl = pl.ds(g * 16, 16)
                cg = pk_v[1, sl]
                gidx_v[sl] = _pad_idx(cg, src_thresh, src_pad)
                rg = pk_v[0, sl]
                loc = rg - c * half_l
                ok = (loc >= 0) & (loc < half_l)
                sidx_v[sl] = jnp.where(ok, loc, half_p)
            pltpu.async_copy(x_hbm.at[gidx_v], xr_v, sem).wait()
            for g in range(K // 16):
                vv = lax.bitcast_convert_type(pk_v[2, pl.ds(g * 16, 16)],
                                              jnp.float32)
                for l in range(16):
                    v = vv[l]
                    e = g * 16 + l
                    for q in range(4):
                        sl = pl.ds(q * 16, 16)
                        xr_v[e, sl] = xr_v[e, sl] * v

            @pl.when(j + 2 < n_chunks)
            def _():
                _pk_copy(j + 2, b).start()
            pltpu.sync_copy(xr_v, acc.at[sidx_v], add=True)

        _pk_copy(0, 0).start()
        _pk_copy(1, 1).start()

        def _chunk(i, carry):
            _process(2 * i, 0)
            _process(2 * i + 1, 1)
            return carry
        lax.fori_loop(0, n_chunks // 2, _chunk, 0)
        if n_chunks % 2:
            _process(n_chunks - 1, 0)
        plsc.subcore_barrier()

        for k in range(zero_chunks):
            r0 = s * tile_rows + k * zero_rows
            pltpu.sync_copy(acc.at[pl.ds(r0, zero_rows)],
                            out_hbm.at[pl.ds(c * half_p + r0, zero_rows)])

    return spmm


_spmm_sub = _make_spmm(NP_ROWS, HALF_L, 88, HALF_L, HALF_P, E_SUB_PAD, 28, 56)
_spmm_soc1 = _make_spmm(N_USERS, N_USERS, 0, SHALF_L, SHALF_P, E_S_PAD, 5, 64)
_spmm_soc2 = _make_spmm(SNP_ROWS, SHALF_L, 120, SHALF_L, SHALF_P, E_S_PAD, 5, 64)


def _pack_edges(rows, cols, vals, e_pad, e):
    rows = jnp.pad(rows.astype(jnp.int32), (0, e_pad - e)).reshape(-1, K)
    cols = jnp.pad(cols.astype(jnp.int32), (0, e_pad - e)).reshape(-1, K)
    vals = lax.bitcast_convert_type(jnp.pad(vals, (0, e_pad - e)),
                                    jnp.int32).reshape(-1, K)
    return jnp.stack([rows, cols, vals], axis=1)

FR = 2000


def _fusion_body(x0, x1, x2, x3, y, w1, b1, w2, b2, w3, b3, t3_ref, ssq_ref):
    x = x0[...] + x1[...] + x2[...] + x3[...]
    yv = y[...]
    c = jnp.concatenate([x, yv, x * yv], axis=1)
    dn = (((1,), (1,)), ((), ()))
    t1 = jnp.tanh(lax.dot_general(c, w1[...], dn,
                                  preferred_element_type=jnp.float32) + b1[...])
    t2 = jnp.tanh(lax.dot_general(t1, w2[...], dn,
                                  preferred_element_type=jnp.float32) + b2[...])
    t3 = lax.dot_general(t2, w3[...], dn,
                         preferred_element_type=jnp.float32) + b3[...]
    t3_ref[...] = t3

    @pl.when(pl.program_id(0) == 0)
    def _():
        ssq_ref[...] = jnp.zeros((1, 1), jnp.float32)
    ssq_ref[...] = ssq_ref[...] + jnp.sum(t3 * t3).reshape(1, 1)


def _fusion(x_parts, y, f1_W, f1_b, f2_W, f2_b, f3_W, f3_b):
    row_spec = pl.BlockSpec((FR, D), lambda i: (i, 0))
    full = lambda shape: pl.BlockSpec(shape, lambda i: (0,) * len(shape))
    t3, ssq = pl.pallas_call(
        _fusion_body,
        grid=(N_USERS // FR,),
        in_specs=[row_spec, row_spec, row_spec, row_spec, row_spec,
                  full((3 * D, 3 * D)), full((1, 3 * D)),
                  full((D, 3 * D)), full((1, D)),
                  full((D, D)), full((1, D))],
        out_specs=[row_spec, pl.BlockSpec((1, 1), lambda i: (0, 0))],
        out_shape=[jax.ShapeDtypeStruct((N_USERS, D), jnp.float32),
                   jax.ShapeDtypeStruct((1, 1), jnp.float32)],
    )(x_parts[0], x_parts[1], x_parts[2], x_parts[3], y,
      f1_W, f1_b.reshape(1, 3 * D), f2_W, f2_b.reshape(1, D),
      f3_W, f3_b.reshape(1, D))
    return t3, ssq


BT = B // (NC * NS)


@functools.partial(
    pl.kernel,
    mesh=_MESH,
    compiler_params=pltpu.CompilerParams(use_tc_tiling_on_sc=False, needs_layout_passes=False),
    out_type=jax.ShapeDtypeStruct((B,), jnp.float32),
    scratch_types=[
        pltpu.VMEM((BT,), jnp.int32),
        pltpu.VMEM((BT,), jnp.int32),
        pltpu.VMEM((BT,), jnp.int32),
        pltpu.VMEM((16,), jnp.float32),
        pltpu.VMEM((BT, D), jnp.float32),
        pltpu.VMEM((BT, D), jnp.float32),
        pltpu.VMEM((BT, D), jnp.float32),
        pltpu.VMEM((BT, D), jnp.float32),
        pltpu.VMEM((BT,), jnp.float32),
        pltpu.SemaphoreType.DMA,
    ],
)
def _final_sc(users_hbm, items_hbm, u_emb, i_emb, t31, t32,
              x10, x11, x12, x13, x20, x21, x22, x23, scal_hbm, out_hbm,
              uid_v, iid_v, nid_v, scal_v, urow_v, irow_v, g1_v, g2_v,
              gam_v, sem):
    c = lax.axis_index("c")
    s = lax.axis_index("s")
    wid = s * NC + c
    base = wid * BT
    pltpu.sync_copy(scal_hbm, scal_v)
    pltpu.sync_copy(users_hbm.at[pl.ds(base, BT)], uid_v)
    pltpu.sync_copy(items_hbm.at[pl.ds(base, BT)], iid_v)
    for g in range(BT // 16):
        sl = pl.ds(g * 16, 16)
        nid = iid_v[sl] + N_USERS
        nid_v[sl] = jnp.where(nid >= HALF_L, nid + 88, nid)
    scal16 = scal_v[pl.ds(0, 16)]
    c0 = scal16[0]
    c1 = scal16[1]
    c2 = scal16[2]
    c3 = scal16[3]

    pltpu.async_copy(u_emb.at[uid_v], urow_v, sem).wait()
    pltpu.async_copy(t31.at[uid_v], g1_v, sem).wait()
    pltpu.async_copy(t32.at[uid_v], g2_v, sem).wait()

    def _ucomb(i, carry):
        e = i // 4
        sl = pl.ds((i % 4) * 16, 16)
        urow_v[e, sl] = (urow_v[e, sl] * c0 + g1_v[e, sl] * c1
                         + g2_v[e, sl] * c2)
        return carry
    lax.fori_loop(0, BT * 4, _ucomb, 0)

    pltpu.async_copy(i_emb.at[iid_v], irow_v, sem).wait()

    def _iscale(i, carry):
        e = i // 4
        sl = pl.ds((i % 4) * 16, 16)
        irow_v[e, sl] = irow_v[e, sl] * c0
        return carry
    lax.fori_loop(0, BT * 4, _iscale, 0)
    for xp in (x10, x11, x12, x13, x20, x21, x22, x23):
        pltpu.async_copy(xp.at[nid_v], g1_v, sem).wait()

        def _iacc(i, carry):
            e = i // 4
            sl = pl.ds((i % 4) * 16, 16)
            irow_v[e, sl] = irow_v[e, sl] + g1_v[e, sl] * c3
            return carry
        lax.fori_loop(0, BT * 4, _iacc, 0)

    iota = lax.iota(jnp.int32, 16)

    def _dot(pg, carry):
        r = pg * 16 + iota
        acc = jnp.zeros((16,), jnp.float32)
        for d in range(D):
            cd = jnp.full((16,), d, jnp.int32)
            uvec = plsc.load_gather(urow_v, [r, cd])
            ivec = plsc.load_gather(irow_v, [r, cd])
            acc = acc + uvec * ivec
        gam_v[pl.ds(pg * 16, 16)] = acc
        return carry
    lax.fori_loop(0, BT // 16, _dot, 0)
    pltpu.sync_copy(gam_v, out_hbm.at[pl.ds(base, BT)])


def _pad50(x):
    z = jnp.zeros((HALF_P - HALF_L, D), jnp.float32)
    return jnp.concatenate([x[:HALF_L], z, x[HALF_L:], z], axis=0)


def kernel(users_, items_, U_emb, I_emb, inter_row, inter_col, inter_val,
           social_row, social_col, social_val, sub_rows, sub_cols, sub_vals,
           fc_W, fc_b, fc2_W, fc2_b, fcp_W, fcp_b,
           f1_W, f1_b, f2_W, f2_b, f3_W, f3_b):
    e0p = _pad50(jnp.concatenate([U_emb, I_emb], axis=0))

    pk_sub = [_pack_edges(sub_rows[p], sub_cols[p], sub_vals[p],
                          E_SUB_PAD, E_SUB) for p in range(P)]
    pk_soc = _pack_edges(social_row, social_col, social_val, E_S_PAD, E_S)

    x1 = [_spmm_sub(pk_sub[p], e0p) for p in range(P)]
    us1 = _spmm_soc1(pk_soc, U_emb)
    us1_u = jnp.concatenate([us1[:SHALF_L], us1[SHALF_P:SHALF_P + SHALF_L]])
    t31, ssq1 = _fusion(x1, us1_u, f1_W, f1_b, f2_W, f2_b, f3_W, f3_b)

    x2 = [_spmm_sub(pk_sub[p], x1[p]) for p in range(P)]
    us2 = _spmm_soc2(pk_soc, us1)
    us2_u = jnp.concatenate([us2[:SHALF_L], us2[SHALF_P:SHALF_P + SHALF_L]])
    t32, ssq2 = _fusion(x2, us2_u, f1_W, f1_b, f2_W, f2_b, f3_W, f3_b)

    inv1 = lax.rsqrt(ssq1[0, 0])
    inv2 = lax.rsqrt(ssq2[0, 0])
    scal = jnp.zeros((16,), jnp.float32)
    scal = scal.at[0].set(4.0 / 3.0)
    scal = scal.at[1].set(inv1 / 3.0)
    scal = scal.at[2].set(inv2 / 3.0)
    scal = scal.at[3].set(1.0 / 3.0)

    gamma = _final_sc(users_.astype(jnp.int32), items_.astype(jnp.int32),
                      U_emb, I_emb, t31, t32,
                      x1[0], x1[1], x1[2], x1[3],
                      x2[0], x2[1], x2[2], x2[3], scal)
    return gamma


# speedup vs baseline: 1.3770x; 1.1240x over previous
import functools

import jax
import jax.numpy as jnp
from jax import lax
from jax.experimental import pallas as pl
from jax.experimental.pallas import tpu as pltpu
from jax.experimental.pallas import tpu_sc as plsc

N_USERS = 10000
M_ITEMS = 40000
N = 50000
D = 64
P = 4
E_S = 160000
E_SUB = 200000
B = 4096

NC = 2
NS = 16
K = 128

HALF_L = 25000
HALF_P = 25088
NP_ROWS = 2 * HALF_P
SHALF_L = 5000
SHALF_P = 5120
SNP_ROWS = 2 * SHALF_P

E_SUB_PAD = 200704
E_S_PAD = 161792

_MESH = plsc.VectorSubcoreMesh(core_axis_name="c", subcore_axis_name="s")


def _pad_idx(col, thresh, pad):
    if pad == 0:
        return col
    return jnp.where(col >= thresh, col + pad, col)


def _make_spmm(n_src, src_thresh, src_pad, half_l, half_p, e_pad,
               zero_chunks, zero_rows):
    e_tile = e_pad // NS
    n_chunks = e_tile // K
    acc_rows = half_p + 8

    @functools.partial(
        pl.kernel,
        mesh=_MESH,
        compiler_params=pltpu.CompilerParams(use_tc_tiling_on_sc=False,
                                             needs_layout_passes=False),
        out_type=jax.ShapeDtypeStruct((2 * half_p, D), jnp.float32),
        scratch_types=[
            pltpu.VMEM((3, K), jnp.int32),
            pltpu.VMEM((3, K), jnp.int32),
            pltpu.VMEM((K,), jnp.int32),
            pltpu.VMEM((K,), jnp.int32),
            pltpu.VMEM((K,), jnp.int32),
            pltpu.VMEM((K,), jnp.int32),
            pltpu.VMEM((K, D), jnp.float32),
            pltpu.VMEM((K, D), jnp.float32),
            pltpu.VMEM((zero_rows, D), jnp.float32),
            pltpu.VMEM_SHARED((acc_rows, D), jnp.float32),
            pltpu.SemaphoreType.DMA,
            pltpu.SemaphoreType.DMA,
            pltpu.SemaphoreType.DMA,
            pltpu.SemaphoreType.DMA,
        ],
    )
    def spmm(pk_hbm, x_hbm, out_hbm,
             pk0, pk1, gi0, gi1, si0, si1, xr0, xr1, zero_v, acc,
             q0, q1, g0, g1):
        c = lax.axis_index("c")
        s = lax.axis_index("s")

        def _zb(i, carry):
            z = jnp.zeros((16,), jnp.float32)
            for q in range(4):
                zero_v[i, pl.ds(q * 16, 16)] = z
            return carry
        lax.fori_loop(0, zero_rows, _zb, 0)
        tile_rows = zero_chunks * zero_rows
        for k in range(zero_chunks):
            pltpu.sync_copy(zero_v, acc.at[pl.ds(s * tile_rows + k * zero_rows,
                                                 zero_rows)])
        plsc.subcore_barrier()

        pk_b = (pk0, pk1)
        gi_b = (gi0, gi1)
        si_b = (si0, si1)
        xr_b = (xr0, xr1)
        qsem = (q0, q1)
        gsem = (g0, g1)

        def _pk_copy(j, b):
            return pltpu.make_async_copy(pk_hbm.at[s * n_chunks + j],
                                         pk_b[b], qsem[b])

        def _idx(j, b):
            pk_v = pk_b[b]
            _pk_copy(j, b).wait()
            for g in range(K // 16):
                sl = pl.ds(g * 16, 16)
                cg = pk_v[1, sl]
                gi_b[b][sl] = _pad_idx(cg, src_thresh, src_pad)
                rg = pk_v[0, sl]
                loc = rg - c * half_l
                ok = (loc >= 0) & (loc < half_l)
                si_b[b][sl] = jnp.where(ok, loc, half_p)

        def _gather(b):
            return pltpu.make_async_copy(x_hbm.at[gi_b[b]], xr_b[b], gsem[b])

        def _process(j, b):
            nb = 1 - b

            @pl.when(j + 1 < n_chunks)
            def _():
                _idx(j + 1, nb)
                _gather(nb).start()
            _gather(b).wait()
            pk_v = pk_b[b]
            xr_v = xr_b[b]
            for g in range(K // 16):
                vv = lax.bitcast_convert_type(pk_v[2, pl.ds(g * 16, 16)],
                                              jnp.float32)
                for l in range(16):
                    v = vv[l]
                    e = g * 16 + l
                    for q in range(4):
                        sl = pl.ds(q * 16, 16)
                        xr_v[e, sl] = xr_v[e, sl] * v

            @pl.when(j + 2 < n_chunks)
            def _():
                _pk_copy(j + 2, b).start()
            pltpu.sync_copy(xr_v, acc.at[si_b[b]], add=True)

        _pk_copy(0, 0).start()
        _pk_copy(1, 1).start()
        _idx(0, 0)
        _gather(0).start()

        def _chunk(i, carry):
            _process(2 * i, 0)
            _process(2 * i + 1, 1)
            return carry
        lax.fori_loop(0, n_chunks // 2, _chunk, 0)
        if n_chunks % 2:
            _process(n_chunks - 1, 0)
        plsc.subcore_barrier()

        for k in range(zero_chunks):
            r0 = s * tile_rows + k * zero_rows
            pltpu.sync_copy(acc.at[pl.ds(r0, zero_rows)],
                            out_hbm.at[pl.ds(c * half_p + r0, zero_rows)])

    return spmm


_spmm_sub = _make_spmm(NP_ROWS, HALF_L, 88, HALF_L, HALF_P, E_SUB_PAD, 28, 56)
_spmm_soc1 = _make_spmm(N_USERS, N_USERS, 0, SHALF_L, SHALF_P, E_S_PAD, 5, 64)
_spmm_soc2 = _make_spmm(SNP_ROWS, SHALF_L, 120, SHALF_L, SHALF_P, E_S_PAD, 5, 64)


def _pack_edges(rows, cols, vals, e_pad, e):
    rows = jnp.pad(rows.astype(jnp.int32), (0, e_pad - e)).reshape(-1, K)
    cols = jnp.pad(cols.astype(jnp.int32), (0, e_pad - e)).reshape(-1, K)
    vals = lax.bitcast_convert_type(jnp.pad(vals, (0, e_pad - e)),
                                    jnp.int32).reshape(-1, K)
    return jnp.stack([rows, cols, vals], axis=1)

FR = 2000


def _fusion_body(x0, x1, x2, x3, y, w1, b1, w2, b2, w3, b3, t3_ref, ssq_ref):
    x = x0[...] + x1[...] + x2[...] + x3[...]
    yv = y[...]
    c = jnp.concatenate([x, yv, x * yv], axis=1)
    dn = (((1,), (1,)), ((), ()))
    t1 = jnp.tanh(lax.dot_general(c, w1[...], dn,
                                  preferred_element_type=jnp.float32) + b1[...])
    t2 = jnp.tanh(lax.dot_general(t1, w2[...], dn,
                                  preferred_element_type=jnp.float32) + b2[...])
    t3 = lax.dot_general(t2, w3[...], dn,
                         preferred_element_type=jnp.float32) + b3[...]
    t3_ref[...] = t3

    @pl.when(pl.program_id(0) == 0)
    def _():
        ssq_ref[...] = jnp.zeros((1, 1), jnp.float32)
    ssq_ref[...] = ssq_ref[...] + jnp.sum(t3 * t3).reshape(1, 1)


def _fusion(x_parts, y, f1_W, f1_b, f2_W, f2_b, f3_W, f3_b):
    row_spec = pl.BlockSpec((FR, D), lambda i: (i, 0))
    full = lambda shape: pl.BlockSpec(shape, lambda i: (0,) * len(shape))
    t3, ssq = pl.pallas_call(
        _fusion_body,
        grid=(N_USERS // FR,),
        in_specs=[row_spec, row_spec, row_spec, row_spec, row_spec,
                  full((3 * D, 3 * D)), full((1, 3 * D)),
                  full((D, 3 * D)), full((1, D)),
                  full((D, D)), full((1, D))],
        out_specs=[row_spec, pl.BlockSpec((1, 1), lambda i: (0, 0))],
        out_shape=[jax.ShapeDtypeStruct((N_USERS, D), jnp.float32),
                   jax.ShapeDtypeStruct((1, 1), jnp.float32)],
    )(x_parts[0], x_parts[1], x_parts[2], x_parts[3], y,
      f1_W, f1_b.reshape(1, 3 * D), f2_W, f2_b.reshape(1, D),
      f3_W, f3_b.reshape(1, D))
    return t3, ssq


BT = B // (NC * NS)


@functools.partial(
    pl.kernel,
    mesh=_MESH,
    compiler_params=pltpu.CompilerParams(use_tc_tiling_on_sc=False, needs_layout_passes=False),
    out_type=jax.ShapeDtypeStruct((B,), jnp.float32),
    scratch_types=[
        pltpu.VMEM((BT,), jnp.int32),
        pltpu.VMEM((BT,), jnp.int32),
        pltpu.VMEM((BT,), jnp.int32),
        pltpu.VMEM((16,), jnp.float32),
        pltpu.VMEM((BT, D), jnp.float32),
        pltpu.VMEM((BT, D), jnp.float32),
        pltpu.VMEM((BT, D), jnp.float32),
        pltpu.VMEM((BT, D), jnp.float32),
        pltpu.VMEM((BT,), jnp.float32),
        pltpu.SemaphoreType.DMA,
    ],
)
def _final_sc(users_hbm, items_hbm, u_emb, i_emb, t31, t32,
              x10, x11, x12, x13, x20, x21, x22, x23, scal_hbm, out_hbm,
              uid_v, iid_v, nid_v, scal_v, urow_v, irow_v, g1_v, g2_v,
              gam_v, sem):
    c = lax.axis_index("c")
    s = lax.axis_index("s")
    wid = s * NC + c
    base = wid * BT
    pltpu.sync_copy(scal_hbm, scal_v)
    pltpu.sync_copy(users_hbm.at[pl.ds(base, BT)], uid_v)
    pltpu.sync_copy(items_hbm.at[pl.ds(base, BT)], iid_v)
    for g in range(BT // 16):
        sl = pl.ds(g * 16, 16)
        nid = iid_v[sl] + N_USERS
        nid_v[sl] = jnp.where(nid >= HALF_L, nid + 88, nid)
    scal16 = scal_v[pl.ds(0, 16)]
    c0 = scal16[0]
    c1 = scal16[1]
    c2 = scal16[2]
    c3 = scal16[3]

    pltpu.async_copy(u_emb.at[uid_v], urow_v, sem).wait()
    pltpu.async_copy(t31.at[uid_v], g1_v, sem).wait()
    pltpu.async_copy(t32.at[uid_v], g2_v, sem).wait()

    def _ucomb(i, carry):
        e = i // 4
        sl = pl.ds((i % 4) * 16, 16)
        urow_v[e, sl] = (urow_v[e, sl] * c0 + g1_v[e, sl] * c1
                         + g2_v[e, sl] * c2)
        return carry
    lax.fori_loop(0, BT * 4, _ucomb, 0)

    pltpu.async_copy(i_emb.at[iid_v], irow_v, sem).wait()

    def _iscale(i, carry):
        e = i // 4
        sl = pl.ds((i % 4) * 16, 16)
        irow_v[e, sl] = irow_v[e, sl] * c0
        return carry
    lax.fori_loop(0, BT * 4, _iscale, 0)
    for xp in (x10, x11, x12, x13, x20, x21, x22, x23):
        pltpu.async_copy(xp.at[nid_v], g1_v, sem).wait()

        def _iacc(i, carry):
            e = i // 4
            sl = pl.ds((i % 4) * 16, 16)
            irow_v[e, sl] = irow_v[e, sl] + g1_v[e, sl] * c3
            return carry
        lax.fori_loop(0, BT * 4, _iacc, 0)

    iota = lax.iota(jnp.int32, 16)

    def _dot(pg, carry):
        r = pg * 16 + iota
        acc = jnp.zeros((16,), jnp.float32)
        for d in range(D):
            cd = jnp.full((16,), d, jnp.int32)
            uvec = plsc.load_gather(urow_v, [r, cd])
            ivec = plsc.load_gather(irow_v, [r, cd])
            acc = acc + uvec * ivec
        gam_v[pl.ds(pg * 16, 16)] = acc
        return carry
    lax.fori_loop(0, BT // 16, _dot, 0)
    pltpu.sync_copy(gam_v, out_hbm.at[pl.ds(base, BT)])


def _pad50(x):
    z = jnp.zeros((HALF_P - HALF_L, D), jnp.float32)
    return jnp.concatenate([x[:HALF_L], z, x[HALF_L:], z], axis=0)


def kernel(users_, items_, U_emb, I_emb, inter_row, inter_col, inter_val,
           social_row, social_col, social_val, sub_rows, sub_cols, sub_vals,
           fc_W, fc_b, fc2_W, fc2_b, fcp_W, fcp_b,
           f1_W, f1_b, f2_W, f2_b, f3_W, f3_b):
    e0p = _pad50(jnp.concatenate([U_emb, I_emb], axis=0))

    pk_sub = [_pack_edges(sub_rows[p], sub_cols[p], sub_vals[p],
                          E_SUB_PAD, E_SUB) for p in range(P)]
    pk_soc = _pack_edges(social_row, social_col, social_val, E_S_PAD, E_S)

    x1 = [_spmm_sub(pk_sub[p], e0p) for p in range(P)]
    us1 = _spmm_soc1(pk_soc, U_emb)
    us1_u = jnp.concatenate([us1[:SHALF_L], us1[SHALF_P:SHALF_P + SHALF_L]])
    t31, ssq1 = _fusion(x1, us1_u, f1_W, f1_b, f2_W, f2_b, f3_W, f3_b)

    x2 = [_spmm_sub(pk_sub[p], x1[p]) for p in range(P)]
    us2 = _spmm_soc2(pk_soc, us1)
    us2_u = jnp.concatenate([us2[:SHALF_L], us2[SHALF_P:SHALF_P + SHALF_L]])
    t32, ssq2 = _fusion(x2, us2_u, f1_W, f1_b, f2_W, f2_b, f3_W, f3_b)

    inv1 = lax.rsqrt(ssq1[0, 0])
    inv2 = lax.rsqrt(ssq2[0, 0])
    scal = jnp.zeros((16,), jnp.float32)
    scal = scal.at[0].set(4.0 / 3.0)
    scal = scal.at[1].set(inv1 / 3.0)
    scal = scal.at[2].set(inv2 / 3.0)
    scal = scal.at[3].set(1.0 / 3.0)

    gamma = _final_sc(users_.astype(jnp.int32), items_.astype(jnp.int32),
                      U_emb, I_emb, t31, t32,
                      x1[0], x1[1], x1[2], x1[3],
                      x2[0], x2[1], x2[2], x2[3], scal)
    return gamma
